# Initial kernel scaffold; baseline (speedup 1.0000x reference)
import jax
import jax.numpy as jnp
from jax.experimental import pallas as pl


def kernel(x, E_class):
    raise NotImplementedError("write your pallas kernel here")


# speedup vs baseline: 2.6734x; 2.6734x over previous
import functools

import jax
import jax.numpy as jnp
import numpy as np
from jax import lax
from jax.experimental import pallas as pl
from jax.experimental.pallas import tpu as pltpu
from jax.experimental.pallas import tpu_sc as plsc

BATCH = 1024
SEQ = 200
EDIM = 128
ATTR = 8
NCLS = SEQ // ATTR
NROWS = BATCH * NCLS

_NC, _NS = 2, 16
_NW = _NC * _NS
_RPW = NROWS // _NW
_CHUNK = 80
_NCHUNK = _RPW // _CHUNK


def _sc_gather_fn():
    mesh = plsc.VectorSubcoreMesh(core_axis_name="c", subcore_axis_name="s")

    @functools.partial(
        pl.kernel,
        mesh=mesh,
        out_type=jax.ShapeDtypeStruct((NROWS, EDIM), jnp.float32),
        scratch_types=[
            pltpu.VMEM((_NCHUNK, _CHUNK), jnp.int32),
            pltpu.VMEM((_RPW, EDIM), jnp.float32),
            pltpu.SemaphoreType.DMA,
        ],
    )
    def sc_gather(idx_hbm, table_hbm, out_hbm, idx_v, rows_v, sem):
        wid = lax.axis_index("s") * _NC + lax.axis_index("c")
        pltpu.sync_copy(idx_hbm.at[wid], idx_v)
        copies = [
            pltpu.async_copy(
                table_hbm.at[idx_v.at[j]],
                rows_v.at[pl.ds(j * _CHUNK, _CHUNK)],
                sem,
            )
            for j in range(_NCHUNK)
        ]
        for c in copies:
            c.wait()
        pltpu.sync_copy(rows_v, out_hbm.at[pl.ds(wid * _RPW, _RPW)])

    return sc_gather


_BB = 32


def _tc_body(xf_ref, g_ref, freq_ref, o_ref):
    xf = xf_ref[...]
    freq = freq_ref[...]
    phase = xf * freq
    shp = phase.shape
    lane = lax.broadcasted_iota(jnp.int32, shp, 3)
    val = jnp.where(lane < EDIM // 2, jnp.sin(phase), jnp.cos(phase))
    sub = lax.broadcasted_iota(jnp.int32, shp, 2)
    o_ref[...] = jnp.where(sub == 0, g_ref[...], val)


def kernel(x, E_class):
    idx = x[:, ::ATTR].reshape(_NW, _NCHUNK, _CHUNK)
    gathered = _sc_gather_fn()(idx, E_class)
    g4 = gathered.reshape(BATCH, NCLS, 1, EDIM)

    xf4 = x.astype(jnp.float32).reshape(BATCH, NCLS, ATTR, 1)
    angles = jnp.arange(EDIM // 2, dtype=jnp.float32)
    freqs = (2.0 ** angles) * np.pi
    freq4 = jnp.concatenate([freqs, freqs]).reshape(1, 1, 1, EDIM)

    out4 = pl.pallas_call(
        _tc_body,
        grid=(BATCH // _BB,),
        in_specs=[
            pl.BlockSpec((_BB, NCLS, ATTR, 1), lambda i: (i, 0, 0, 0)),
            pl.BlockSpec((_BB, NCLS, 1, EDIM), lambda i: (i, 0, 0, 0)),
            pl.BlockSpec((1, 1, 1, EDIM), lambda i: (0, 0, 0, 0)),
        ],
        out_specs=pl.BlockSpec((_BB, NCLS, ATTR, EDIM), lambda i: (i, 0, 0, 0)),
        out_shape=jax.ShapeDtypeStruct((BATCH, NCLS, ATTR, EDIM), jnp.float32),
    )(xf4, g4, freq4)

    return out4.reshape(BATCH, SEQ, EDIM)


# speedup vs baseline: 3.9653x; 1.4832x over previous
import functools

import jax
import jax.numpy as jnp
import numpy as np
from jax import lax
from jax.experimental import pallas as pl
from jax.experimental.pallas import tpu as pltpu
from jax.experimental.pallas import tpu_sc as plsc

BATCH = 1024
SEQ = 200
EDIM = 128
ATTR = 8
NCLS = SEQ // ATTR
NROWS = BATCH * NCLS

_NC, _NS = 2, 16
_NW = _NC * _NS
_RPW = NROWS // _NW
_CHUNK = 80
_NCHUNK = _RPW // _CHUNK


def _sc_gather_fn():
    mesh = plsc.VectorSubcoreMesh(core_axis_name="c", subcore_axis_name="s")

    @functools.partial(
        pl.kernel,
        mesh=mesh,
        out_type=jax.ShapeDtypeStruct((NROWS, EDIM), jnp.float32),
        scratch_types=[
            pltpu.VMEM((_NCHUNK, _CHUNK), jnp.int32),
            pltpu.VMEM((_RPW, EDIM), jnp.float32),
            pltpu.SemaphoreType.DMA,
        ],
    )
    def sc_gather(idx_hbm, table_hbm, out_hbm, idx_v, rows_v, sem):
        wid = lax.axis_index("s") * _NC + lax.axis_index("c")
        pltpu.sync_copy(idx_hbm.at[wid], idx_v)
        copies = [
            pltpu.async_copy(
                table_hbm.at[idx_v.at[j]],
                rows_v.at[pl.ds(j * _CHUNK, _CHUNK)],
                sem,
            )
            for j in range(_NCHUNK)
        ]
        for c in copies:
            c.wait()
        pltpu.sync_copy(rows_v, out_hbm.at[pl.ds(wid * _RPW, _RPW)])

    return sc_gather


_BB = 32


_HB = _BB // 2


def _tc_body(xf_ref, g_ref, freq_ref, o_ref):
    shp = (_HB, NCLS, ATTR, EDIM)
    lane = lax.broadcasted_iota(jnp.int32, shp, 3)
    low = lane < EDIM // 2
    xf = xf_ref[...]
    xa = jnp.broadcast_to(xf[:_HB], shp)
    xb = jnp.broadcast_to(xf[_HB:], shp)
    freq = freq_ref[...]
    phase = jnp.where(low, xa, xb) * freq
    s = jnp.sin(phase)
    c = jnp.cos(phase)
    out_a = jnp.where(low, s, pltpu.roll(c, EDIM // 2, 3))
    out_b = jnp.where(low, pltpu.roll(s, EDIM // 2, 3), c)
    sub = lax.broadcasted_iota(jnp.int32, shp, 2)
    g = g_ref[...]
    o_ref[:_HB] = jnp.where(sub == 0, g[:_HB], out_a)
    o_ref[_HB:] = jnp.where(sub == 0, g[_HB:], out_b)


def kernel(x, E_class):
    idx = x[:, ::ATTR].reshape(_NW, _NCHUNK, _CHUNK)
    gathered = _sc_gather_fn()(idx, E_class)
    g4 = gathered.reshape(BATCH, NCLS, 1, EDIM)

    xf4 = x.astype(jnp.float32).reshape(BATCH, NCLS, ATTR, 1)
    angles = jnp.arange(EDIM // 2, dtype=jnp.float32)
    freqs = (2.0 ** angles) * np.pi
    freq4 = jnp.concatenate([freqs, freqs]).reshape(1, 1, 1, EDIM)

    out4 = pl.pallas_call(
        _tc_body,
        grid=(BATCH // _BB,),
        in_specs=[
            pl.BlockSpec((_BB, NCLS, ATTR, 1), lambda i: (i, 0, 0, 0)),
            pl.BlockSpec((_BB, NCLS, 1, EDIM), lambda i: (i, 0, 0, 0)),
            pl.BlockSpec((1, 1, 1, EDIM), lambda i: (0, 0, 0, 0)),
        ],
        out_specs=pl.BlockSpec((_BB, NCLS, ATTR, EDIM), lambda i: (i, 0, 0, 0)),
        out_shape=jax.ShapeDtypeStruct((BATCH, NCLS, ATTR, EDIM), jnp.float32),
    )(xf4, g4, freq4)

    return out4.reshape(BATCH, SEQ, EDIM)


# speedup vs baseline: 10.2040x; 2.5734x over previous
import functools

import jax
import jax.numpy as jnp
import numpy as np
from jax import lax
from jax.experimental import pallas as pl
from jax.experimental.pallas import tpu as pltpu
from jax.experimental.pallas import tpu_sc as plsc

BATCH = 1024
SEQ = 200
EDIM = 128
ATTR = 8
NCLS = SEQ // ATTR
NROWS = BATCH * NCLS

_NC, _NS = 2, 16
_NW = _NC * _NS
_RPW = NROWS // _NW
_CHUNK = 80
_NCHUNK = _RPW // _CHUNK


def _sc_gather_fn():
    mesh = plsc.VectorSubcoreMesh(core_axis_name="c", subcore_axis_name="s")

    @functools.partial(
        pl.kernel,
        mesh=mesh,
        out_type=jax.ShapeDtypeStruct((NROWS, EDIM), jnp.float32),
        scratch_types=[
            pltpu.VMEM((_NCHUNK, _CHUNK), jnp.int32),
            pltpu.VMEM((_RPW, EDIM), jnp.float32),
        ]
        + [pltpu.SemaphoreType.DMA] * _NCHUNK
        + [pltpu.SemaphoreType.DMA],
    )
    def sc_gather(idx_hbm, table_hbm, out_hbm, idx_v, rows_v, *sems):
        gsems, ssem = sems[:_NCHUNK], sems[_NCHUNK]
        wid = lax.axis_index("s") * _NC + lax.axis_index("c")
        pltpu.sync_copy(idx_hbm.at[wid], idx_v)
        gathers = [
            pltpu.async_copy(
                table_hbm.at[idx_v.at[j]],
                rows_v.at[pl.ds(j * _CHUNK, _CHUNK)],
                gsems[j],
            )
            for j in range(_NCHUNK)
        ]
        scatters = []
        for j in range(_NCHUNK):
            gathers[j].wait()
            scatters.append(
                pltpu.async_copy(
                    rows_v.at[pl.ds(j * _CHUNK, _CHUNK)],
                    out_hbm.at[pl.ds(wid * _RPW + j * _CHUNK, _CHUNK)],
                    ssem,
                )
            )
        for sc in scatters:
            sc.wait()

    return sc_gather


_BB = 32
_RESTART = 16


def _tc_body(xf_ref, g_ref, o_ref):
    w = xf_ref[...].astype(jnp.float32) * np.float32(np.pi)
    s = jnp.sin(w)
    c = jnp.cos(w)
    slist, clist = [s], [c]
    for k in range(1, EDIM // 2):
        if k % _RESTART == 0:
            ph = w * np.float32(2.0 ** k)
            s = jnp.sin(ph)
            c = jnp.cos(ph)
        else:
            s, c = 2.0 * s * c, c * c - s * s
        slist.append(s)
        clist.append(c)
    S = jnp.stack(slist + clist, axis=0)
    r0 = lax.broadcasted_iota(jnp.int32, (EDIM, EDIM), 0)
    r1 = lax.broadcasted_iota(jnp.int32, (EDIM, EDIM), 1)
    ident = (r0 == r1).astype(jnp.float32)
    O = lax.dot_general(
        S, ident, (((0,), (0,)), ((), ())),
        precision=lax.Precision.DEFAULT,
    )
    O5 = O.reshape(_BB, NCLS, ATTR, EDIM)
    sub = lax.broadcasted_iota(jnp.int32, (_BB, NCLS, ATTR, EDIM), 2)
    o_ref[...] = jnp.where(sub == 0, g_ref[...], O5)


def kernel(x, E_class):
    idx = x[:, ::ATTR].reshape(_NW, _NCHUNK, _CHUNK)
    gathered = _sc_gather_fn()(idx, E_class)
    g4 = gathered.reshape(BATCH, NCLS, 1, EDIM)

    out4 = pl.pallas_call(
        _tc_body,
        grid=(BATCH // _BB,),
        in_specs=[
            pl.BlockSpec((_BB, SEQ), lambda i: (i, 0)),
            pl.BlockSpec((_BB, NCLS, 1, EDIM), lambda i: (i, 0, 0, 0)),
        ],
        out_specs=pl.BlockSpec((_BB, NCLS, ATTR, EDIM), lambda i: (i, 0, 0, 0)),
        out_shape=jax.ShapeDtypeStruct((BATCH, NCLS, ATTR, EDIM), jnp.float32),
    )(x, g4)

    return out4.reshape(BATCH, SEQ, EDIM)


# speedup vs baseline: 10.3439x; 1.0137x over previous
import functools

import jax
import jax.numpy as jnp
import numpy as np
from jax import lax
from jax.experimental import pallas as pl
from jax.experimental.pallas import tpu as pltpu
from jax.experimental.pallas import tpu_sc as plsc

BATCH = 1024
SEQ = 200
EDIM = 128
ATTR = 8
NCLS = SEQ // ATTR
NROWS = BATCH * NCLS

_NC, _NS = 2, 16
_NW = _NC * _NS
_RPW = NROWS // _NW
_CHUNK = 80
_NCHUNK = _RPW // _CHUNK


def _sc_gather_fn(nrows):
    mesh = plsc.VectorSubcoreMesh(core_axis_name="c", subcore_axis_name="s")
    rpw = nrows // _NW
    nchunk = rpw // _CHUNK

    @functools.partial(
        pl.kernel,
        mesh=mesh,
        out_type=jax.ShapeDtypeStruct((nrows, EDIM), jnp.float32),
        scratch_types=[
            pltpu.VMEM((nchunk, _CHUNK), jnp.int32),
            pltpu.VMEM((rpw, EDIM), jnp.float32),
            pltpu.SemaphoreType.DMA,
        ],
    )
    def sc_gather(idx_hbm, table_hbm, out_hbm, idx_v, rows_v, sem):
        wid = lax.axis_index("s") * _NC + lax.axis_index("c")
        pltpu.sync_copy(idx_hbm.at[wid], idx_v)
        copies = [
            pltpu.async_copy(
                table_hbm.at[idx_v.at[j]],
                rows_v.at[pl.ds(j * _CHUNK, _CHUNK)],
                sem,
            )
            for j in range(nchunk)
        ]
        for c in copies:
            c.wait()
        pltpu.sync_copy(rows_v, out_hbm.at[pl.ds(wid * rpw, rpw)])

    return sc_gather


_BB = 32
_RESTART = 16


def _tc_body(xf_ref, g_ref, o_ref):
    w = xf_ref[...].astype(jnp.float32) * np.float32(np.pi)
    s = jnp.sin(w)
    c = jnp.cos(w)
    slist, clist = [s], [c]
    for k in range(1, EDIM // 2):
        if k % _RESTART == 0:
            ph = w * np.float32(2.0 ** k)
            s = jnp.sin(ph)
            c = jnp.cos(ph)
        else:
            s, c = 2.0 * s * c, c * c - s * s
        slist.append(s)
        clist.append(c)
    S = jnp.stack(slist + clist, axis=0)
    r0 = lax.broadcasted_iota(jnp.int32, (EDIM, EDIM), 0)
    r1 = lax.broadcasted_iota(jnp.int32, (EDIM, EDIM), 1)
    ident = (r0 == r1).astype(jnp.float32)
    O = lax.dot_general(
        S, ident, (((0,), (0,)), ((), ())),
        precision=lax.Precision.DEFAULT,
    )
    O5 = O.reshape(_BB, NCLS, ATTR, EDIM)
    sub = lax.broadcasted_iota(jnp.int32, (_BB, NCLS, ATTR, EDIM), 2)
    o_ref[...] = jnp.where(sub == 0, g_ref[...], O5)


_HBATCH = BATCH // 2
_HGRID = _HBATCH // _BB


def kernel(x, E_class):
    hc = _HBATCH * NCLS
    sc = _sc_gather_fn(hc)
    idx1 = x[:_HBATCH, ::ATTR].reshape(_NW, hc // _NW // _CHUNK, _CHUNK)
    idx2 = x[_HBATCH:, ::ATTR].reshape(_NW, hc // _NW // _CHUNK, _CHUNK)
    g1 = sc(idx1, E_class).reshape(_HBATCH, NCLS, 1, EDIM)
    g2 = sc(idx2, E_class).reshape(_HBATCH, NCLS, 1, EDIM)

    out_shape = jax.ShapeDtypeStruct((BATCH, NCLS, ATTR, EDIM), jnp.float32)
    o1 = pl.pallas_call(
        _tc_body,
        grid=(_HGRID,),
        in_specs=[
            pl.BlockSpec((_BB, SEQ), lambda i: (i, 0)),
            pl.BlockSpec((_BB, NCLS, 1, EDIM), lambda i: (i, 0, 0, 0)),
        ],
        out_specs=pl.BlockSpec((_BB, NCLS, ATTR, EDIM), lambda i: (i, 0, 0, 0)),
        out_shape=out_shape,
    )(x, g1)

    def _tc_body2(xf_ref, g_ref, prev_ref, o_ref):
        _tc_body(xf_ref, g_ref, o_ref)

    out4 = pl.pallas_call(
        _tc_body2,
        grid=(_HGRID,),
        in_specs=[
            pl.BlockSpec((_BB, SEQ), lambda i: (i + _HGRID, 0)),
            pl.BlockSpec((_BB, NCLS, 1, EDIM), lambda i: (i, 0, 0, 0)),
            pl.BlockSpec(memory_space=pltpu.MemorySpace.HBM),
        ],
        out_specs=pl.BlockSpec((_BB, NCLS, ATTR, EDIM), lambda i: (i + _HGRID, 0, 0, 0)),
        out_shape=out_shape,
        input_output_aliases={2: 0},
    )(x, g2, o1)

    return out4.reshape(BATCH, SEQ, EDIM)
